# baseline (speedup 1.0000x reference)
import functools

import jax
import jax.numpy as jnp
from jax import lax
from jax.experimental import pallas as pl
from jax.experimental.pallas import tpu as pltpu
from jax.experimental.pallas import tpu_sc as plsc

D = 64
NFIELD = 3
B = 16384
ROW_W = NFIELD * D

_info = plsc.get_sparse_core_info()
_NC, _NS = _info.num_cores, _info.num_subcores
NW = _NC * _NS
BPW = B // NW
HALF = BPW // 2
L = 16
NGROUP = HALF // L
AHEAD = 4


def _body(uid, iid, cid, wu, wi, wc, out, idx_v, ruf, cat_v, sem):
    wid = lax.axis_index("s") * _NC + lax.axis_index("c")
    base = wid * BPW

    for f, idx_hbm in enumerate((uid, iid, cid)):
        pltpu.sync_copy(
            idx_hbm.at[pl.ds(base, BPW)], idx_v.at[pl.ds(f * BPW, BPW)]
        )

    def drain_rows(n):
        def one(_, x):
            pltpu.make_async_copy(
                wu.at[pl.ds(0, 1)], ruf.at[pl.ds(0, 1)], sem
            ).wait()
            return x

        lax.fori_loop(0, n, one, 0)

    for h in range(2):
        for f, table in enumerate((wu, wi, wc)):

            def gather16(g, x, table=table, f=f, h=h):
                @pl.when(g >= AHEAD)
                def _drain():
                    drain_rows(L)

                v = idx_v[pl.ds(f * BPW + h * HALF + g * L, L)]
                for k in range(L):
                    pltpu.async_copy(
                        table.at[pl.ds(v[k], 1)],
                        ruf.at[pl.ds(g * L + k, 1)],
                        sem,
                    )
                return x

            lax.fori_loop(0, NGROUP, gather16, 0)
            drain_rows(AHEAD * L)

            def assemble(r, x, f=f):
                for q in range(D // L):
                    cat_v[r, pl.ds(f * D + q * L, L)] = ruf[r, pl.ds(q * L, L)]
                return x

            lax.fori_loop(0, HALF, assemble, 0)

        pltpu.sync_copy(cat_v, out.at[pl.ds(base + h * HALF, HALF)])


@jax.jit
def kernel(user_id, item_id, cat_id, W_user, W_item, W_cat):
    mesh = plsc.VectorSubcoreMesh(core_axis_name="c", subcore_axis_name="s")
    run = functools.partial(
        pl.kernel,
        out_type=jax.ShapeDtypeStruct((B, ROW_W), jnp.float32),
        scratch_types=[
            pltpu.VMEM((NFIELD * BPW,), jnp.int32),
            pltpu.VMEM((HALF, D), jnp.float32),
            pltpu.VMEM((HALF, ROW_W), jnp.float32),
            pltpu.SemaphoreType.DMA,
        ],
        mesh=mesh,
    )(_body)
    return run(
        user_id.astype(jnp.int32),
        item_id.astype(jnp.int32),
        cat_id.astype(jnp.int32),
        W_user,
        W_item,
        W_cat,
    )
